# Initial kernel scaffold; baseline (speedup 1.0000x reference)
import jax
import jax.numpy as jnp
from jax.experimental import pallas as pl


def kernel(x, perm):
    raise NotImplementedError("write your pallas kernel here")


# speedup vs baseline: 1.5058x; 1.5058x over previous
import jax
import jax.numpy as jnp
from jax import lax
from jax.experimental import pallas as pl
from jax.experimental.pallas import tpu as pltpu


_BATCH = 8192
_CH = 4096
_LANE = 128


def _transpose_body(x_ref, o_ref):
    o_ref[...] = x_ref[...].T


def _transpose(x):
    bi, bj = 1024, 512
    return pl.pallas_call(
        _transpose_body,
        grid=(_BATCH // bi, _CH // bj),
        in_specs=[pl.BlockSpec((bi, bj), lambda i, j: (i, j))],
        out_specs=pl.BlockSpec((bj, bi), lambda i, j: (j, i)),
        out_shape=jax.ShapeDtypeStruct((_CH, _BATCH), jnp.float32),
    )(x)


def _gather_body(perm_ref, xt_ref, o_ref, scratch, sem):
    t = pl.program_id(0)

    def issue(l, _):
        row = perm_ref[t * _LANE + l]
        pltpu.make_async_copy(
            xt_ref.at[pl.ds(row, 1), :], scratch.at[pl.ds(l, 1), :], sem
        ).start()
        return _

    lax.fori_loop(0, _LANE, issue, None)

    def wait(l, _):
        row = perm_ref[t * _LANE + l]
        pltpu.make_async_copy(
            xt_ref.at[pl.ds(row, 1), :], scratch.at[pl.ds(l, 1), :], sem
        ).wait()
        return _

    lax.fori_loop(0, _LANE, wait, None)
    o_ref[...] = scratch[...].T


def kernel(x, perm):
    perm32 = perm.astype(jnp.int32)
    xt = _transpose(x)
    grid_spec = pltpu.PrefetchScalarGridSpec(
        num_scalar_prefetch=1,
        grid=(_CH // _LANE,),
        in_specs=[pl.BlockSpec(memory_space=pl.ANY)],
        out_specs=pl.BlockSpec((_BATCH, _LANE), lambda t, perm_ref: (0, t)),
        scratch_shapes=[
            pltpu.VMEM((_LANE, _BATCH), jnp.float32),
            pltpu.SemaphoreType.DMA,
        ],
    )
    return pl.pallas_call(
        _gather_body,
        grid_spec=grid_spec,
        out_shape=jax.ShapeDtypeStruct((_BATCH, _CH), jnp.float32),
    )(perm32, xt)


# speedup vs baseline: 2.0718x; 1.3759x over previous
import jax
import jax.numpy as jnp
from jax import lax
from jax.experimental import pallas as pl
from jax.experimental.pallas import tpu as pltpu


_BATCH = 8192
_CH = 4096
_LANE = 128
_NT = _CH // _LANE


def _transpose_body(x_ref, o_ref):
    o_ref[...] = x_ref[...].T


def _transpose(x):
    bi, bj = 1024, 512
    return pl.pallas_call(
        _transpose_body,
        grid=(_BATCH // bi, _CH // bj),
        in_specs=[pl.BlockSpec((bi, bj), lambda i, j: (i, j))],
        out_specs=pl.BlockSpec((bj, bi), lambda i, j: (j, i)),
        out_shape=jax.ShapeDtypeStruct((_CH, _BATCH), jnp.float32),
    )(x)


def _issue_tile(perm_ref, xt_ref, scratch, sem, t, slot):
    def issue(l, _):
        row = perm_ref[t * _LANE + l]
        pltpu.make_async_copy(
            xt_ref.at[pl.ds(row, 1), :],
            scratch.at[slot, pl.ds(l, 1), :],
            sem.at[slot],
        ).start()
        return _

    lax.fori_loop(0, _LANE, issue, None)


def _gather_body(perm_ref, xt_ref, o_ref, scratch, sem):
    t = pl.program_id(0)
    slot = lax.rem(t, 2)
    nxt = lax.rem(t + 1, 2)

    @pl.when(t == 0)
    def _():
        _issue_tile(perm_ref, xt_ref, scratch, sem, 0, 0)

    @pl.when(t + 1 < _NT)
    def _():
        _issue_tile(perm_ref, xt_ref, scratch, sem, t + 1, nxt)

    pltpu.make_async_copy(
        xt_ref.at[pl.ds(0, _LANE), :], scratch.at[slot], sem.at[slot]
    ).wait()
    o_ref[...] = scratch[slot].T


def kernel(x, perm):
    perm32 = perm.astype(jnp.int32)
    xt = _transpose(x)
    grid_spec = pltpu.PrefetchScalarGridSpec(
        num_scalar_prefetch=1,
        grid=(_NT,),
        in_specs=[pl.BlockSpec(memory_space=pl.ANY)],
        out_specs=pl.BlockSpec((_BATCH, _LANE), lambda t, perm_ref: (0, t)),
        scratch_shapes=[
            pltpu.VMEM((2, _LANE, _BATCH), jnp.float32),
            pltpu.SemaphoreType.DMA((2,)),
        ],
    )
    return pl.pallas_call(
        _gather_body,
        grid_spec=grid_spec,
        out_shape=jax.ShapeDtypeStruct((_BATCH, _CH), jnp.float32),
    )(perm32, xt)


# speedup vs baseline: 2.2568x; 1.0893x over previous
import jax
import jax.numpy as jnp
from jax import lax
from jax.experimental import pallas as pl
from jax.experimental.pallas import tpu as pltpu


_BATCH = 8192
_CH = 4096
_LANE = 128
_NT = _CH // _LANE


def _transpose_body(x_ref, o_ref):
    o_ref[...] = x_ref[...].T


def _transpose(x):
    bj = 256
    return pl.pallas_call(
        _transpose_body,
        grid=(_CH // bj,),
        in_specs=[pl.BlockSpec((_BATCH, bj), lambda j: (0, j))],
        out_specs=pl.BlockSpec((bj, _BATCH), lambda j: (j, 0)),
        out_shape=jax.ShapeDtypeStruct((_CH, _BATCH), jnp.float32),
    )(x)


def _issue_tile(perm_ref, xt_ref, scratch, sem, t, slot):
    def issue(l, _):
        row = perm_ref[t * _LANE + l]
        pltpu.make_async_copy(
            xt_ref.at[pl.ds(row, 1), :],
            scratch.at[slot, pl.ds(l, 1), :],
            sem.at[slot],
        ).start()
        return _

    lax.fori_loop(0, _LANE, issue, None)


def _gather_body(perm_ref, xt_ref, o_ref, scratch, sem):
    t = pl.program_id(0)
    slot = lax.rem(t, 2)
    nxt = lax.rem(t + 1, 2)

    @pl.when(t == 0)
    def _():
        _issue_tile(perm_ref, xt_ref, scratch, sem, 0, 0)

    @pl.when(t + 1 < _NT)
    def _():
        _issue_tile(perm_ref, xt_ref, scratch, sem, t + 1, nxt)

    pltpu.make_async_copy(
        xt_ref.at[pl.ds(0, _LANE), :], scratch.at[slot], sem.at[slot]
    ).wait()
    o_ref[...] = scratch[slot].T


def kernel(x, perm):
    perm32 = perm.astype(jnp.int32)
    xt = _transpose(x)
    grid_spec = pltpu.PrefetchScalarGridSpec(
        num_scalar_prefetch=1,
        grid=(_NT,),
        in_specs=[pl.BlockSpec(memory_space=pl.ANY)],
        out_specs=pl.BlockSpec((_BATCH, _LANE), lambda t, perm_ref: (0, t)),
        scratch_shapes=[
            pltpu.VMEM((2, _LANE, _BATCH), jnp.float32),
            pltpu.SemaphoreType.DMA((2,)),
        ],
    )
    return pl.pallas_call(
        _gather_body,
        grid_spec=grid_spec,
        out_shape=jax.ShapeDtypeStruct((_BATCH, _CH), jnp.float32),
    )(perm32, xt)
